# baseline (speedup 1.0000x reference)
import functools

import jax
import jax.numpy as jnp
from jax import lax
from jax.experimental import pallas as pl
from jax.experimental.pallas import tpu as pltpu
from jax.experimental.pallas import tpu_sc as plsc

N = 1024
K = 128
NB = N - K
NA = N - 8
KA = K - 8

NC = 2
NS = 16
NW = NC * NS
RPW = N // NW


@functools.partial(
    pl.kernel,
    mesh=plsc.VectorSubcoreMesh(core_axis_name="c", subcore_axis_name="s"),
    out_type=[
        jax.ShapeDtypeStruct((N, N), jnp.float32),
        jax.ShapeDtypeStruct((NA, N), jnp.float32),
        jax.ShapeDtypeStruct((8, N), jnp.float32),
    ],
    scratch_types=[
        pltpu.VMEM((8, 16, 16), jnp.float32),
        pltpu.VMEM((RPW, N), jnp.float32),
        pltpu.VMEM((RPW, N), jnp.float32),
        pltpu.VMEM((8, N), jnp.float32),
        pltpu.SemaphoreType.DMA,
    ],
)
def _sc_build_sparse(o_hbm, right_hbm, fw_hbm, mw_hbm,
                     o_v, buf1, buf2, buf3, sem):
    f32 = jnp.float32
    w = lax.axis_index("s") * NC + lax.axis_index("c")
    base = w * RPW
    zero = jnp.zeros((16,), f32)
    lane = lax.iota(jnp.int32, 16)

    pltpu.sync_copy(o_hbm, o_v)

    def zero_rows(buf, nrows):
        def body(r, carry):
            for c in range(N // 16):
                buf[r, pl.ds(c * 16, 16)] = zero
            return carry
        lax.fori_loop(0, nrows, body, 0)

    zero_rows(buf1, RPW)

    @pl.when(w < 4)
    def _():
        def body(j, carry):
            rg = base + j
            buf1[j, pl.ds((rg // 16) * 16, 16)] = o_v[rg // 16, rg % 16, :]
            return carry
        lax.fori_loop(0, RPW, body, 0)

    @pl.when(w >= 4)
    def _():
        def body(j, carry):
            rg = base + j
            oh = jnp.where(lane == rg % 16, 1.0, 0.0).astype(f32)
            buf1[j, pl.ds((rg // 16) * 16, 16)] = oh
            return carry
        lax.fori_loop(0, RPW, body, 0)

    cp1 = pltpu.async_copy(buf1, right_hbm.at[pl.ds(base, RPW)], sem)

    zero_rows(buf2, RPW)

    def fw_body(j, carry):
        pg = base + j

        @pl.when(pg < KA)
        def _():
            l = pg // 15
            buf2[j, pl.ds(l * 16, 16)] = o_v[l, pg % 15 + 1, :]

        @pl.when((pg >= KA) & (pg < NA))
        def _():
            col = pg + 8
            oh = jnp.where(lane == col % 16, 1.0, 0.0).astype(f32)
            buf2[j, pl.ds((col // 16) * 16, 16)] = oh

        return carry

    lax.fori_loop(0, RPW, fw_body, 0)

    @pl.when(w < NW - 1)
    def _():
        pltpu.sync_copy(buf2, fw_hbm.at[pl.ds(base, RPW)])

    @pl.when(w == NW - 1)
    def _():
        pltpu.sync_copy(buf2.at[pl.ds(0, RPW - 8)],
                        fw_hbm.at[pl.ds(base, RPW - 8)])
        zero_rows(buf3, 8)
        for l in range(8):
            buf3[l, pl.ds(l * 16, 16)] = o_v[l, 0, :]
        pltpu.sync_copy(buf3, mw_hbm)

    cp1.wait()


def _strip_kernel(a_top_ref, a_colsL_ref, o_ref,
                  dtop_ref, artop_ref, dleft_ref, arleft_ref,
                  fctl_ref, fctr_ref, fcleft_ref, mc_ref):
    f32 = jnp.float32
    a_top = a_top_ref[...]
    a_left = a_colsL_ref[...][K:, :]
    o = o_ref[...]

    o128 = o.reshape(K, 16)
    x = jnp.concatenate([o128] * 8, axis=1)
    rr = jax.lax.broadcasted_iota(jnp.int32, (K, K), 0)
    cc = jax.lax.broadcasted_iota(jnp.int32, (K, K), 1)
    b = jnp.where((rr // 16) == (cc // 16), x, 0.0)

    gp = jax.lax.broadcasted_iota(jnp.int32, (KA, K), 0)
    gq = jax.lax.broadcasted_iota(jnp.int32, (KA, K), 1)
    g = ((gp // 15) * 16 + (gp % 15) + 1 == gq).astype(f32)
    ep = jax.lax.broadcasted_iota(jnp.int32, (8, K), 0)
    eq = jax.lax.broadcasted_iota(jnp.int32, (8, K), 1)
    e = (ep * 16 == eq).astype(f32)

    dot = lambda u, v: jnp.dot(u, v, preferred_element_type=f32)

    t = dot(b, a_top)
    m = dot(t[:, :K], b.T)
    t896 = dot(a_left, b.T)

    ri = jax.lax.broadcasted_iota(jnp.int32, (K, N), 0)
    ci = jax.lax.broadcasted_iota(jnp.int32, (K, N), 1)
    act_r = (ri % 16) != 0
    act_c = (ci >= K) | ((ci % 16) != 0)
    af_top = jnp.concatenate([m, t[:, K:]], axis=1)
    d_top = jnp.where((ri == ci) | (act_r & act_c), af_top, 0.0)
    dtop_ref[...] = d_top

    colmask = ((jax.lax.broadcasted_iota(jnp.int32, (NB, K), 1) % 16) != 0)
    d_left = jnp.where(colmask, t896, 0.0)
    dleft_ref[...] = d_left

    s = dot(b.T, d_top)
    artop_ref[...] = jnp.concatenate([dot(s[:, :K], b), s[:, K:]], axis=1)
    arleft_ref[...] = dot(d_left, b)

    fctl_ref[...] = dot(dot(g, m), g.T)
    fctr_ref[...] = dot(g, t[:, K:])
    fcleft_ref[...] = dot(t896, g.T)

    eme = dot(dot(e, m), e.T)
    i8 = jax.lax.broadcasted_iota(jnp.int32, (8, 8), 0)
    j8 = jax.lax.broadcasted_iota(jnp.int32, (8, 8), 1)
    mc_ref[...] = jnp.where(i8 == j8, eme, 0.0)


def _assemble_kernel(a_ref, apeek_ref, dtop_ref, artop_ref,
                     dleft_ref, arleft_ref, fctl_ref, fctr_ref,
                     fcleft_ref, fcpeek_ref,
                     d_ref, ar_ref, fc_ref):
    i = pl.program_id(0)

    @pl.when(i == 0)
    def _():
        d_ref[...] = dtop_ref[...]
        ar_ref[...] = artop_ref[...]
        fc_top = jnp.concatenate([fctl_ref[...], fctr_ref[...]], axis=1)
        fc_bot = jnp.concatenate(
            [fcpeek_ref[...], apeek_ref[...][:, K:]], axis=1)
        fc_ref[...] = jnp.concatenate([fc_top, fc_bot], axis=0)

    @pl.when(i > 0)
    def _():
        bottom = a_ref[...][:, K:]
        d_ref[...] = jnp.concatenate([dleft_ref[...], bottom], axis=1)
        ar_ref[...] = jnp.concatenate([arleft_ref[...], bottom], axis=1)
        fcleft_win = jnp.concatenate(
            [fcleft_ref[...][8:], fcpeek_ref[...]], axis=0)
        a_win = jnp.concatenate(
            [a_ref[...][8:, K:], apeek_ref[...][:, K:]], axis=0)
        fc_ref[...] = jnp.concatenate([fcleft_win, a_win], axis=1)


def kernel(A_dense, O, rot_rows, rot_cols, wav_idx, act_idx):
    f32 = jnp.float32
    sds = jax.ShapeDtypeStruct

    right, fw, mw = _sc_build_sparse(O)

    strips = pl.pallas_call(
        _strip_kernel,
        grid=(1,),
        in_specs=[
            pl.BlockSpec((K, N), lambda i: (0, 0)),
            pl.BlockSpec((N, K), lambda i: (0, 0)),
            pl.BlockSpec((8, 16, 16), lambda i: (0, 0, 0)),
        ],
        out_specs=[
            pl.BlockSpec((K, N), lambda i: (0, 0)),
            pl.BlockSpec((K, N), lambda i: (0, 0)),
            pl.BlockSpec((NB, K), lambda i: (0, 0)),
            pl.BlockSpec((NB, K), lambda i: (0, 0)),
            pl.BlockSpec((KA, KA), lambda i: (0, 0)),
            pl.BlockSpec((KA, NB), lambda i: (0, 0)),
            pl.BlockSpec((NB, KA), lambda i: (0, 0)),
            pl.BlockSpec((8, 8), lambda i: (0, 0)),
        ],
        out_shape=[
            sds((K, N), f32),
            sds((K, N), f32),
            sds((NB, K), f32),
            sds((NB, K), f32),
            sds((KA, KA), f32),
            sds((KA, NB), f32),
            sds((NB, KA), f32),
            sds((8, 8), f32),
        ],
    )(A_dense, A_dense, O)
    d_top, ar_top, d_left, ar_left, fctl, fctr, fcleft, mc = strips

    d, a_rec, fc = pl.pallas_call(
        _assemble_kernel,
        grid=(8,),
        in_specs=[
            pl.BlockSpec((K, N), lambda i: (i, 0)),
            pl.BlockSpec((8, N),
                         lambda i: (jnp.minimum(16 * (i + 1), 127), 0)),
            pl.BlockSpec((K, N), lambda i: (0, 0)),
            pl.BlockSpec((K, N), lambda i: (0, 0)),
            pl.BlockSpec((K, K), lambda i: (jnp.maximum(i - 1, 0), 0)),
            pl.BlockSpec((K, K), lambda i: (jnp.maximum(i - 1, 0), 0)),
            pl.BlockSpec((KA, KA), lambda i: (0, 0)),
            pl.BlockSpec((KA, NB), lambda i: (0, 0)),
            pl.BlockSpec((K, KA), lambda i: (jnp.maximum(i - 1, 0), 0)),
            pl.BlockSpec((8, KA),
                         lambda i: (jnp.minimum(16 * i, 111), 0)),
        ],
        out_specs=[
            pl.BlockSpec((K, N), lambda i: (i, 0)),
            pl.BlockSpec((K, N), lambda i: (i, 0)),
            pl.BlockSpec((K, NA), lambda i: (i, 0)),
        ],
        out_shape=[
            sds((N, N), f32), sds((N, N), f32), sds((NA, NA), f32),
        ],
    )(A_dense, A_dense, d_top, ar_top, d_left, ar_left,
      fctl, fctr, fcleft, fcleft)

    return (a_rec, right, d, mc, fc, mw, fw)


# speedup vs baseline: 1.1024x; 1.1024x over previous
import functools

import jax
import jax.numpy as jnp
from jax import lax
from jax.experimental import pallas as pl
from jax.experimental.pallas import tpu as pltpu
from jax.experimental.pallas import tpu_sc as plsc

N = 1024
K = 128
NB = N - K
NA = N - 8
KA = K - 8

NC = 2
NS = 16
NW = NC * NS
RPW = N // NW


@functools.cache
def _sc_builder():
    return functools.partial(
        pl.kernel,
        mesh=plsc.VectorSubcoreMesh(core_axis_name="c", subcore_axis_name="s"),
        out_type=[
            jax.ShapeDtypeStruct((N, N), jnp.float32),
            jax.ShapeDtypeStruct((NA, N), jnp.float32),
            jax.ShapeDtypeStruct((8, N), jnp.float32),
        ],
        scratch_types=[
            pltpu.VMEM((8, 16, 16), jnp.float32),
            pltpu.VMEM((RPW, N), jnp.float32),
            pltpu.VMEM((8, N), jnp.float32),
        ],
    )(_sc_body)


def _sc_body(o_hbm, right_hbm, fw_hbm, mw_hbm, o_v, buf, buf3):
    f32 = jnp.float32
    w = lax.axis_index("s") * NC + lax.axis_index("c")
    base = w * RPW
    zero = jnp.zeros((16,), f32)
    lane = lax.iota(jnp.int32, 16)

    @pl.when((w < 4) | (w == NW - 1))
    def _():
        pltpu.sync_copy(o_hbm, o_v)

    def zero_rows(buf_, nrows):
        def body(r, carry):
            def inner(c, carry2):
                for u in range(16):
                    buf_[r, pl.ds((c * 16 + u) * 16, 16)] = zero
                return carry2
            return lax.fori_loop(0, 4, inner, carry)
        lax.fori_loop(0, nrows, body, 0)

    zero_rows(buf, RPW)

    @pl.when(w < 4)
    def _():
        def body(j, carry):
            rg = base + j
            buf[j, pl.ds((rg // 16) * 16, 16)] = o_v[rg // 16, rg % 16, :]
            return carry
        lax.fori_loop(0, RPW, body, 0)

    @pl.when(w >= 4)
    def _():
        def body(j, carry):
            rg = base + j
            oh = jnp.where(lane == rg % 16, 1.0, 0.0).astype(f32)
            buf[j, pl.ds((rg // 16) * 16, 16)] = oh
            return carry
        lax.fori_loop(0, RPW, body, 0)

    pltpu.sync_copy(buf, right_hbm.at[pl.ds(base, RPW)])

    def clear_body(j, carry):
        rg = base + j
        buf[j, pl.ds((rg // 16) * 16, 16)] = zero
        return carry
    lax.fori_loop(0, RPW, clear_body, 0)

    def fw_body(j, carry):
        pg = base + j

        @pl.when(pg < KA)
        def _():
            l = pg // 15
            buf[j, pl.ds(l * 16, 16)] = o_v[l, pg % 15 + 1, :]

        @pl.when((pg >= KA) & (pg < NA))
        def _():
            col = pg + 8
            oh = jnp.where(lane == col % 16, 1.0, 0.0).astype(f32)
            buf[j, pl.ds((col // 16) * 16, 16)] = oh

        return carry

    lax.fori_loop(0, RPW, fw_body, 0)

    @pl.when(w < NW - 1)
    def _():
        pltpu.sync_copy(buf, fw_hbm.at[pl.ds(base, RPW)])

    @pl.when(w == NW - 1)
    def _():
        pltpu.sync_copy(buf.at[pl.ds(0, RPW - 8)],
                        fw_hbm.at[pl.ds(base, RPW - 8)])
        zero_rows(buf3, 8)
        def mw_body(l, carry):
            buf3[l, pl.ds(l * 16, 16)] = o_v[l, 0, :]
            return carry
        lax.fori_loop(0, 8, mw_body, 0)
        pltpu.sync_copy(buf3, mw_hbm)


def _strip_kernel(a_top_ref, a_colsL_ref, o_ref,
                  dtop_ref, artop_ref, dleft_ref, arleft_ref,
                  fctl_ref, fctr_ref, fcleft_ref, mc_ref):
    f32 = jnp.float32
    a_top = a_top_ref[...]
    a_left = a_colsL_ref[...][K:, :]
    o = o_ref[...]

    o128 = o.reshape(K, 16)
    x = jnp.concatenate([o128] * 8, axis=1)
    rr = jax.lax.broadcasted_iota(jnp.int32, (K, K), 0)
    cc = jax.lax.broadcasted_iota(jnp.int32, (K, K), 1)
    b = jnp.where((rr // 16) == (cc // 16), x, 0.0)

    gp = jax.lax.broadcasted_iota(jnp.int32, (KA, K), 0)
    gq = jax.lax.broadcasted_iota(jnp.int32, (KA, K), 1)
    g = ((gp // 15) * 16 + (gp % 15) + 1 == gq).astype(f32)
    ep = jax.lax.broadcasted_iota(jnp.int32, (8, K), 0)
    eq = jax.lax.broadcasted_iota(jnp.int32, (8, K), 1)
    e = (ep * 16 == eq).astype(f32)

    dot = lambda u, v: jnp.dot(u, v, preferred_element_type=f32)

    t = dot(b, a_top)
    m = dot(t[:, :K], b.T)
    t896 = dot(a_left, b.T)

    ri = jax.lax.broadcasted_iota(jnp.int32, (K, N), 0)
    ci = jax.lax.broadcasted_iota(jnp.int32, (K, N), 1)
    act_r = (ri % 16) != 0
    act_c = (ci >= K) | ((ci % 16) != 0)
    af_top = jnp.concatenate([m, t[:, K:]], axis=1)
    d_top = jnp.where((ri == ci) | (act_r & act_c), af_top, 0.0)
    dtop_ref[...] = d_top

    colmask = ((jax.lax.broadcasted_iota(jnp.int32, (NB, K), 1) % 16) != 0)
    d_left = jnp.where(colmask, t896, 0.0)
    z128 = jnp.zeros((K, K), f32)
    dleft_ref[...] = jnp.concatenate([z128, d_left], axis=0)

    s = dot(b.T, d_top)
    artop_ref[...] = jnp.concatenate([dot(s[:, :K], b), s[:, K:]], axis=1)
    arleft_ref[...] = jnp.concatenate([z128, dot(d_left, b)], axis=0)

    fctl_ref[...] = dot(dot(g, m), g.T)
    fctr_ref[...] = dot(g, t[:, K:])
    fcleft_ref[...] = jnp.concatenate(
        [jnp.zeros((KA, KA), f32), dot(t896, g.T), jnp.zeros((8, KA), f32)],
        axis=0)

    eme = dot(dot(e, m), e.T)
    i8 = jax.lax.broadcasted_iota(jnp.int32, (8, 8), 0)
    j8 = jax.lax.broadcasted_iota(jnp.int32, (8, 8), 1)
    mc_ref[...] = jnp.where(i8 == j8, eme, 0.0)


BR = 256


def _assemble_kernel(a_ref, apeek_ref, dtop_ref, artop_ref,
                     dleft_ref, arleft_ref, fctl_ref, fctr_ref,
                     fcleft_ref,
                     d_ref, ar_ref, fc_ref):
    i = pl.program_id(0)

    @pl.when(i == 0)
    def _():
        a_bot = a_ref[...][K:, K:]
        d_ref[...] = jnp.concatenate(
            [dtop_ref[...],
             jnp.concatenate([dleft_ref[...][K:], a_bot], axis=1)], axis=0)
        ar_ref[...] = jnp.concatenate(
            [artop_ref[...],
             jnp.concatenate([arleft_ref[...][K:], a_bot], axis=1)], axis=0)
        fc_top = jnp.concatenate([fctl_ref[...], fctr_ref[...]], axis=1)
        a_win = jnp.concatenate(
            [a_ref[...][K:, K:], apeek_ref[...][:, K:]], axis=0)
        fc_bot = jnp.concatenate([fcleft_ref[...][KA:], a_win], axis=1)
        fc_ref[...] = jnp.concatenate([fc_top, fc_bot], axis=0)

    @pl.when(i > 0)
    def _():
        bottom = a_ref[...][:, K:]
        d_ref[...] = jnp.concatenate([dleft_ref[...], bottom], axis=1)
        ar_ref[...] = jnp.concatenate([arleft_ref[...], bottom], axis=1)
        a_win = jnp.concatenate(
            [a_ref[...][8:, K:], apeek_ref[...][:, K:]], axis=0)
        fc_ref[...] = jnp.concatenate([fcleft_ref[...], a_win], axis=1)


def kernel(A_dense, O, rot_rows, rot_cols, wav_idx, act_idx):
    f32 = jnp.float32
    sds = jax.ShapeDtypeStruct

    right, fw, mw = _sc_builder()(O)

    strips = pl.pallas_call(
        _strip_kernel,
        grid=(1,),
        in_specs=[
            pl.BlockSpec((K, N), lambda i: (0, 0)),
            pl.BlockSpec((N, K), lambda i: (0, 0)),
            pl.BlockSpec((8, 16, 16), lambda i: (0, 0, 0)),
        ],
        out_specs=[
            pl.BlockSpec((K, N), lambda i: (0, 0)),
            pl.BlockSpec((K, N), lambda i: (0, 0)),
            pl.BlockSpec((N, K), lambda i: (0, 0)),
            pl.BlockSpec((N, K), lambda i: (0, 0)),
            pl.BlockSpec((KA, KA), lambda i: (0, 0)),
            pl.BlockSpec((KA, NB), lambda i: (0, 0)),
            pl.BlockSpec((N, KA), lambda i: (0, 0)),
            pl.BlockSpec((8, 8), lambda i: (0, 0)),
        ],
        out_shape=[
            sds((K, N), f32),
            sds((K, N), f32),
            sds((N, K), f32),
            sds((N, K), f32),
            sds((KA, KA), f32),
            sds((KA, NB), f32),
            sds((N, KA), f32),
            sds((8, 8), f32),
        ],
    )(A_dense, A_dense, O)
    d_top, ar_top, d_left, ar_left, fctl, fctr, fcleft, mc = strips

    d, a_rec, fc = pl.pallas_call(
        _assemble_kernel,
        grid=(4,),
        in_specs=[
            pl.BlockSpec((BR, N), lambda i: (i, 0)),
            pl.BlockSpec((8, N),
                         lambda i: (jnp.minimum(32 * (i + 1), 127), 0)),
            pl.BlockSpec((K, N), lambda i: (0, 0)),
            pl.BlockSpec((K, N), lambda i: (0, 0)),
            pl.BlockSpec((BR, K), lambda i: (i, 0)),
            pl.BlockSpec((BR, K), lambda i: (i, 0)),
            pl.BlockSpec((KA, KA), lambda i: (0, 0)),
            pl.BlockSpec((KA, NB), lambda i: (0, 0)),
            pl.BlockSpec((BR, KA), lambda i: (i, 0)),
        ],
        out_specs=[
            pl.BlockSpec((BR, N), lambda i: (i, 0)),
            pl.BlockSpec((BR, N), lambda i: (i, 0)),
            pl.BlockSpec((BR, NA), lambda i: (i, 0)),
        ],
        out_shape=[
            sds((N, N), f32), sds((N, N), f32), sds((NA, NA), f32),
        ],
    )(A_dense, A_dense, d_top, ar_top, d_left, ar_left,
      fctl, fctr, fcleft)

    return (a_rec, right, d, mc, fc, mw, fw)


# speedup vs baseline: 2.8226x; 2.5605x over previous
import functools

import jax
import jax.numpy as jnp
from jax import lax
from jax.experimental import pallas as pl
from jax.experimental.pallas import tpu as pltpu
from jax.experimental.pallas import tpu_sc as plsc

N = 1024
K = 128
NB = N - K
NA = N - 8
KA = K - 8
BR = 256


def _iota2(shape, dim):
    return jax.lax.broadcasted_iota(jnp.int32, shape, dim)


def _constants(o):
    f32 = jnp.float32
    o128 = o.reshape(K, 16)
    x = jnp.concatenate([o128] * 8, axis=1)
    b = jnp.where((_iota2((K, K), 0) // 16) == (_iota2((K, K), 1) // 16),
                  x, 0.0)
    gp, gq = _iota2((KA, K), 0), _iota2((KA, K), 1)
    g = ((gp // 15) * 16 + (gp % 15) + 1 == gq).astype(f32)
    e = (_iota2((8, K), 0) * 16 == _iota2((8, K), 1)).astype(f32)
    return b, g, e


def _main_kernel(a_ref, apeek_ref, o_ref,
                 d_ref, ar_ref, fc_ref, right_ref, fw_ref, mc_ref, mw_ref):
    f32 = jnp.float32
    i = pl.program_id(0)
    dot = lambda u, v: jnp.dot(u, v, preferred_element_type=f32)
    b, g, e = _constants(o_ref[...])
    ablk = a_ref[...]
    arows = jnp.concatenate([ablk[8:], apeek_ref[...]], axis=0)

    fcl = dot(dot(arows[:, :K], b.T), g.T)

    rowid = BR * i + _iota2((BR, N), 0)
    colid = _iota2((BR, N), 1)

    @pl.when(i == 0)
    def _():
        a_top = ablk[:K, :]
        t = dot(b, a_top)
        m = dot(t[:, :K], b.T)
        ri, ci = _iota2((K, N), 0), _iota2((K, N), 1)
        act_r = (ri % 16) != 0
        act_c = (ci >= K) | ((ci % 16) != 0)
        af_top = jnp.concatenate([m, t[:, K:]], axis=1)
        d_top = jnp.where((ri == ci) | (act_r & act_c), af_top, 0.0)
        s = dot(b.T, d_top)
        ar_top = jnp.concatenate([dot(s[:, :K], b), s[:, K:]], axis=1)

        t_bot = dot(ablk[K:, :K], b.T)
        cmask = (_iota2((K, K), 1) % 16) != 0
        d_bl = jnp.where(cmask, t_bot, 0.0)
        a_br = ablk[K:, K:]
        d_ref[...] = jnp.concatenate(
            [d_top, jnp.concatenate([d_bl, a_br], axis=1)], axis=0)
        ar_ref[...] = jnp.concatenate(
            [ar_top, jnp.concatenate([dot(d_bl, b), a_br], axis=1)], axis=0)

        fc_top = jnp.concatenate([dot(dot(g, m), g.T), dot(g, t[:, K:])],
                                 axis=1)
        fc_bot = jnp.concatenate([fcl[KA:], arows[KA:, K:]], axis=1)
        fc_ref[...] = jnp.concatenate([fc_top, fc_bot], axis=0)

        right_ref[...] = jnp.concatenate(
            [jnp.concatenate([b, jnp.zeros((K, NB), f32)], axis=1),
             (rowid[K:] == colid[K:]).astype(f32)], axis=0)
        fw_top = jnp.concatenate([dot(g, b), jnp.zeros((KA, NB), f32)],
                                 axis=1)
        fw_bot = (colid[:136] == _iota2((136, N), 0) + KA + 8).astype(f32)
        fw_ref[...] = jnp.concatenate([fw_top, fw_bot], axis=0)

        eme = dot(dot(e, m), e.T)
        mc_ref[...] = jnp.where(_iota2((8, 8), 0) == _iota2((8, 8), 1),
                                eme, 0.0)
        mw_ref[...] = jnp.concatenate(
            [dot(e, b), jnp.zeros((8, NB), f32)], axis=1)

    @pl.when(i > 0)
    def _():
        t_blk = dot(ablk[:, :K], b.T)
        cmask = (_iota2((BR, K), 1) % 16) != 0
        d_l = jnp.where(cmask, t_blk, 0.0)
        a_r = ablk[:, K:]
        d_ref[...] = jnp.concatenate([d_l, a_r], axis=1)
        ar_ref[...] = jnp.concatenate([dot(d_l, b), a_r], axis=1)
        fc_ref[...] = jnp.concatenate([fcl, arows[:, K:]], axis=1)
        right_ref[...] = (rowid == colid).astype(f32)
        fw_ref[...] = (colid == rowid + 8).astype(f32)


def kernel(A_dense, O, rot_rows, rot_cols, wav_idx, act_idx):
    f32 = jnp.float32
    sds = jax.ShapeDtypeStruct

    d, a_rec, fc, right, fw, mc, mw = pl.pallas_call(
        _main_kernel,
        grid=(4,),
        in_specs=[
            pl.BlockSpec((BR, N), lambda i: (i, 0)),
            pl.BlockSpec((8, N),
                         lambda i: (jnp.minimum(32 * (i + 1), 127), 0)),
            pl.BlockSpec((8, 16, 16), lambda i: (0, 0, 0)),
        ],
        out_specs=[
            pl.BlockSpec((BR, N), lambda i: (i, 0)),
            pl.BlockSpec((BR, N), lambda i: (i, 0)),
            pl.BlockSpec((BR, NA), lambda i: (i, 0)),
            pl.BlockSpec((BR, N), lambda i: (i, 0)),
            pl.BlockSpec((BR, N), lambda i: (i, 0)),
            pl.BlockSpec((8, 8), lambda i: (0, 0)),
            pl.BlockSpec((8, N), lambda i: (0, 0)),
        ],
        out_shape=[
            sds((N, N), f32),
            sds((N, N), f32),
            sds((NA, NA), f32),
            sds((N, N), f32),
            sds((NA, N), f32),
            sds((8, 8), f32),
            sds((8, N), f32),
        ],
    )(A_dense, A_dense, O)

    return (a_rec, right, d, mc, fc, mw, fw)
